# Initial kernel scaffold; baseline (speedup 1.0000x reference)
import jax
import jax.numpy as jnp
from jax.experimental import pallas as pl


def kernel(hidden_states, topk_weights, topk_ids, wi_0, wi_1, wo):
    raise NotImplementedError("write your pallas kernel here")


# speedup vs baseline: 4.0380x; 4.0380x over previous
import jax
import jax.numpy as jnp
from jax.experimental import pallas as pl
from jax.experimental.pallas import tpu as pltpu

_T, _H, _F, _E, _K = 2048, 1024, 1024, 8, 2
_TM = 1024


def _dense_body(tw_ref, ids_ref, x_ref, w0_ref, w1_ref, wo_ref, out_ref):
    e = pl.program_id(1)
    x = x_ref[...]
    h0 = jnp.dot(x, w0_ref[0], preferred_element_type=jnp.float32)
    h1 = jnp.dot(x, w1_ref[0], preferred_element_type=jnp.float32)
    inter = (h0 * jax.nn.sigmoid(h0)) * h1
    y = jnp.dot(inter, wo_ref[0], preferred_element_type=jnp.float32)
    w = jnp.sum(jnp.where(ids_ref[...] == e, tw_ref[...], 0.0), axis=1, keepdims=True)
    contrib = y * w

    @pl.when(e == 0)
    def _init():
        out_ref[...] = contrib

    @pl.when(e > 0)
    def _acc():
        out_ref[...] += contrib


def kernel(hidden_states, topk_weights, topk_ids, wi_0, wi_1, wo):
    ids = topk_ids.astype(jnp.int32)
    return pl.pallas_call(
        _dense_body,
        grid=(_T // _TM, _E),
        in_specs=[
            pl.BlockSpec((_TM, _K), lambda t, e: (t, 0)),
            pl.BlockSpec((_TM, _K), lambda t, e: (t, 0)),
            pl.BlockSpec((_TM, _H), lambda t, e: (t, 0)),
            pl.BlockSpec((1, _H, _F), lambda t, e: (e, 0, 0)),
            pl.BlockSpec((1, _H, _F), lambda t, e: (e, 0, 0)),
            pl.BlockSpec((1, _F, _H), lambda t, e: (e, 0, 0)),
        ],
        out_specs=pl.BlockSpec((_TM, _H), lambda t, e: (t, 0)),
        out_shape=jax.ShapeDtypeStruct((_T, _H), jnp.float32),
    )(topk_weights, ids, hidden_states, wi_0, wi_1, wo)
